# initial kernel scaffold (unmeasured)
import jax
import jax.numpy as jnp
from jax import lax
from jax.experimental import pallas as pl
from jax.experimental.pallas import tpu as pltpu

N_DEV = 4


def kernel(x, w_mat, scale_x, scale_w):
    m, k_per = x.shape
    _, n = w_mat.shape

    def _mm(a, b):
        return jax.lax.dot_general(
            a.astype(jnp.bfloat16), b.astype(jnp.bfloat16),
            dimension_numbers=(((1,), (0,)), ((), ())),
            preferred_element_type=jnp.float32,
        )

    def body(x_ref, w_ref, sx_ref, sw_ref, out_ref,
             xg, wg, x_send, x_recv, w_send, w_recv):
        my = lax.axis_index("i")
        left = (my - 1) % N_DEV
        right = (my + 1) % N_DEV

        barrier = pltpu.get_barrier_semaphore()
        for nbr in (left, right):
            pl.semaphore_signal(
                barrier, inc=1,
                device_id=(nbr,), device_id_type=pl.DeviceIdType.MESH,
            )
        pl.semaphore_wait(barrier, 2)

        for h in range(N_DEV - 1):
            x_rdma = pltpu.make_async_remote_copy(
                src_ref=x_ref if h == 0 else xg.at[h - 1],
                dst_ref=xg.at[h],
                send_sem=x_send.at[h], recv_sem=x_recv.at[h],
                device_id=(right,), device_id_type=pl.DeviceIdType.MESH,
            )
            w_rdma = pltpu.make_async_remote_copy(
                src_ref=w_ref if h == 0 else wg.at[h - 1],
                dst_ref=wg.at[h],
                send_sem=w_send.at[h], recv_sem=w_recv.at[h],
                device_id=(left,), device_id_type=pl.DeviceIdType.MESH,
            )
            x_rdma.start()
            w_rdma.start()
            if h == 0:
                out_ref[...] = _mm(x_ref[...], w_ref[...])
            if h == 2:
                out_ref[...] += _mm(xg[1], wg[1])
            x_rdma.wait()
            w_rdma.wait()

        out_ref[...] += _mm(xg[0], wg[2])
        out_ref[...] += _mm(xg[2], wg[0])
        out_ref[...] *= sx_ref[0] * sw_ref[0]

    return pl.pallas_call(
        body,
        out_shape=jax.ShapeDtypeStruct((m, n), jnp.float32),
        in_specs=[
            pl.BlockSpec(memory_space=pltpu.VMEM),
            pl.BlockSpec(memory_space=pltpu.VMEM),
            pl.BlockSpec(memory_space=pltpu.SMEM),
            pl.BlockSpec(memory_space=pltpu.SMEM),
        ],
        out_specs=pl.BlockSpec(memory_space=pltpu.VMEM),
        scratch_shapes=[
            pltpu.VMEM((N_DEV - 1, m, k_per), x.dtype),
            pltpu.VMEM((N_DEV - 1, k_per, n), w_mat.dtype),
            pltpu.SemaphoreType.DMA((N_DEV - 1,)),
            pltpu.SemaphoreType.DMA((N_DEV - 1,)),
            pltpu.SemaphoreType.DMA((N_DEV - 1,)),
            pltpu.SemaphoreType.DMA((N_DEV - 1,)),
        ],
        compiler_params=pltpu.CompilerParams(collective_id=0),
    )(x, w_mat, scale_x, scale_w)


# baseline (device time: 226074 ns/iter reference)
import jax
import jax.numpy as jnp
from jax import lax
from jax.experimental import pallas as pl
from jax.experimental.pallas import tpu as pltpu

N_DEV = 4


def kernel(x, w_mat, scale_x, scale_w):
    x = x.astype(jnp.float8_e5m2)
    w_mat = w_mat.astype(jnp.float8_e5m2)
    m, k_per = x.shape
    _, n = w_mat.shape
    blk = m // N_DEV

    def _mm(a, b):
        return jax.lax.dot_general(
            a, b,
            dimension_numbers=(((1,), (0,)), ((), ())),
            preferred_element_type=jnp.float32,
        )

    def body(x_ref, w_ref, sx_ref, sw_ref, out_ref,
             xg, wg, acc, x_send, x_recv, w_send, w_recv, copy_sems):
        my = lax.axis_index("i")
        left = (my - 1) % N_DEV
        right = (my + 1) % N_DEV

        barrier = pltpu.get_barrier_semaphore()
        for nbr in (left, right):
            pl.semaphore_signal(
                barrier, inc=1,
                device_id=(nbr,), device_id_type=pl.DeviceIdType.MESH,
            )
        pl.semaphore_wait(barrier, 2)

        for h in range(N_DEV - 1):
            x_rdma = pltpu.make_async_remote_copy(
                src_ref=x_ref if h == 0 else xg.at[h - 1],
                dst_ref=xg.at[h],
                send_sem=x_send.at[h], recv_sem=x_recv.at[h],
                device_id=(right,), device_id_type=pl.DeviceIdType.MESH,
            )
            w_rdma = pltpu.make_async_remote_copy(
                src_ref=w_ref if h == 0 else wg.at[h - 1],
                dst_ref=wg.at[h],
                send_sem=w_send.at[h], recv_sem=w_recv.at[h],
                device_id=(left,), device_id_type=pl.DeviceIdType.MESH,
            )
            x_rdma.start()
            w_rdma.start()
            x_rdma.wait()
            w_rdma.wait()

        scale = sx_ref[0] * sw_ref[0]
        copies = [None, None]
        for r in range(N_DEV):
            slot = r % 2
            if copies[slot] is not None:
                copies[slot].wait()
            rows = pl.ds(r * blk, blk)
            a = (_mm(x_ref[rows, :], w_ref[...])
                 + _mm(xg[1, rows, :], wg[1])
                 + _mm(xg[0, rows, :], wg[2])
                 + _mm(xg[2, rows, :], wg[0]))
            acc[slot] = a * scale
            cp = pltpu.make_async_copy(
                acc.at[slot], out_ref.at[rows, :], copy_sems.at[slot]
            )
            cp.start()
            copies[slot] = cp
        for cp in copies:
            cp.wait()

    return pl.pallas_call(
        body,
        out_shape=jax.ShapeDtypeStruct((m, n), jnp.float32),
        in_specs=[
            pl.BlockSpec(memory_space=pltpu.VMEM),
            pl.BlockSpec(memory_space=pltpu.VMEM),
            pl.BlockSpec(memory_space=pltpu.SMEM),
            pl.BlockSpec(memory_space=pltpu.SMEM),
        ],
        out_specs=pl.BlockSpec(memory_space=pl.ANY),
        scratch_shapes=[
            pltpu.VMEM((N_DEV - 1, m, k_per), jnp.float8_e5m2),
            pltpu.VMEM((N_DEV - 1, k_per, n), jnp.float8_e5m2),
            pltpu.VMEM((2, blk, n), jnp.float32),
            pltpu.SemaphoreType.DMA((N_DEV - 1,)),
            pltpu.SemaphoreType.DMA((N_DEV - 1,)),
            pltpu.SemaphoreType.DMA((N_DEV - 1,)),
            pltpu.SemaphoreType.DMA((N_DEV - 1,)),
            pltpu.SemaphoreType.DMA((2,)),
        ],
        compiler_params=pltpu.CompilerParams(
            collective_id=0,
            vmem_limit_bytes=63 * 1024 * 1024,
        ),
    )(x, w_mat, scale_x, scale_w)


# device time: 190509 ns/iter; 1.1867x vs baseline; 1.1867x over previous
import jax
import jax.numpy as jnp
from jax import lax
from jax.experimental import pallas as pl
from jax.experimental.pallas import tpu as pltpu

N_DEV = 4
RB = 8
SUB = 4


def kernel(x, w_mat, scale_x, scale_w):
    x = x.astype(jnp.float8_e5m2)
    w_mat = w_mat.astype(jnp.float8_e5m2)
    m, k_per = x.shape
    _, n = w_mat.shape
    rblk = m // RB
    sblk = m // SUB

    def _mm(a, b):
        return jax.lax.dot_general(
            a, b,
            dimension_numbers=(((1,), (0,)), ((), ())),
            preferred_element_type=jnp.float32,
        )

    def body(x_ref, w_ref, sx_ref, sw_ref, out_ref,
             xg, wg, accf, stage,
             x_send, x_recv, w_send, w_recv, xs_send, xs_recv, copy_sems):
        my = lax.axis_index("i")
        left = (my - 1) % N_DEV
        right = (my + 1) % N_DEV

        def rcopy(src, dst, ssem, rsem, dev):
            return pltpu.make_async_remote_copy(
                src_ref=src, dst_ref=dst, send_sem=ssem, recv_sem=rsem,
                device_id=(dev,), device_id_type=pl.DeviceIdType.MESH,
            )

        def acc_pair(xpc, wpc, first=False):
            for rb in range(RB):
                rows = pl.ds(rb * rblk, rblk)
                prod = _mm(xpc[rows, :], wpc[...])
                if first:
                    accf[rows, :] = prod.astype(jnp.bfloat16)
                else:
                    accf[rows, :] = (
                        accf[rows, :].astype(jnp.float32) + prod
                    ).astype(jnp.bfloat16)

        barrier = pltpu.get_barrier_semaphore()
        for nbr in (left, right):
            pl.semaphore_signal(
                barrier, inc=1,
                device_id=(nbr,), device_id_type=pl.DeviceIdType.MESH,
            )
        pl.semaphore_wait(barrier, 2)

        h0x = rcopy(x_ref, xg.at[0], x_send.at[0], x_recv.at[0], right)
        h0w = rcopy(w_ref, wg.at[0], w_send.at[0], w_recv.at[0], left)
        h0x.start()
        h0w.start()
        acc_pair(x_ref, w_ref, first=True)

        h0w.wait()
        h1w = rcopy(wg.at[0], wg.at[1], w_send.at[1], w_recv.at[1], left)
        h1w.start()
        h0x.wait()
        h1x = rcopy(xg.at[0], xg.at[1], x_send.at[1], x_recv.at[1], right)
        h1x.start()
        h1w.wait()
        h2w = rcopy(wg.at[1], wg.at[2], w_send.at[2], w_recv.at[2], left)
        h2w.start()
        h2w.wait()

        acc_pair(xg.at[0], wg.at[2])

        h1x.wait()
        subs = []
        for s in range(SUB):
            srows = pl.ds(s * sblk, sblk)
            r = rcopy(xg.at[1, srows, :], xg.at[2, srows, :],
                      xs_send.at[s], xs_recv.at[s], right)
            r.start()
            subs.append(r)

        acc_pair(xg.at[1], wg.at[1])

        scale = sx_ref[0] * sw_ref[0]
        copies = [None, None]
        for s in range(SUB):
            subs[s].wait()
            for half in range(sblk // rblk):
                rbi = s * (sblk // rblk) + half
                rows = pl.ds(rbi * rblk, rblk)
                val = (
                    accf[rows, :].astype(jnp.float32)
                    + _mm(xg[2, rows, :], wg[0])
                ) * scale
                slot = rbi % 2
                if copies[slot] is not None:
                    copies[slot].wait()
                stage[slot, :, :] = val
                cp = pltpu.make_async_copy(
                    stage.at[slot], out_ref.at[rows, :], copy_sems.at[slot]
                )
                cp.start()
                copies[slot] = cp
        for cp in copies:
            cp.wait()

    return pl.pallas_call(
        body,
        out_shape=jax.ShapeDtypeStruct((m, n), jnp.float32),
        in_specs=[
            pl.BlockSpec(memory_space=pltpu.VMEM),
            pl.BlockSpec(memory_space=pltpu.VMEM),
            pl.BlockSpec(memory_space=pltpu.SMEM),
            pl.BlockSpec(memory_space=pltpu.SMEM),
        ],
        out_specs=pl.BlockSpec(memory_space=pl.ANY),
        scratch_shapes=[
            pltpu.VMEM((N_DEV - 1, m, k_per), jnp.float8_e5m2),
            pltpu.VMEM((N_DEV - 1, k_per, n), jnp.float8_e5m2),
            pltpu.VMEM((m, n), jnp.bfloat16),
            pltpu.VMEM((2, rblk, n), jnp.float32),
            pltpu.SemaphoreType.DMA((2,)),
            pltpu.SemaphoreType.DMA((2,)),
            pltpu.SemaphoreType.DMA((N_DEV - 1,)),
            pltpu.SemaphoreType.DMA((N_DEV - 1,)),
            pltpu.SemaphoreType.DMA((SUB,)),
            pltpu.SemaphoreType.DMA((SUB,)),
            pltpu.SemaphoreType.DMA((2,)),
        ],
        compiler_params=pltpu.CompilerParams(
            collective_id=0,
            vmem_limit_bytes=63 * 1024 * 1024,
        ),
    )(x, w_mat, scale_x, scale_w)


# device time: 175064 ns/iter; 1.2914x vs baseline; 1.0882x over previous
import jax
import jax.numpy as jnp
from jax import lax
from jax.experimental import pallas as pl
from jax.experimental.pallas import tpu as pltpu

N_DEV = 4
RB = 8
SUB = 4
XC = 8


def kernel(x, w_mat, scale_x, scale_w):
    m, k_per = x.shape
    _, n = w_mat.shape
    rblk = m // RB
    sblk = m // SUB
    xcb = m // XC

    def _mm(a, b):
        return jax.lax.dot_general(
            a, b,
            dimension_numbers=(((1,), (0,)), ((), ())),
            preferred_element_type=jnp.float32,
        )

    def body(x_hbm, w_hbm, sx_ref, sw_ref, out_ref,
             xl, wl, xg, wg, accf, stage, xstg, wstg,
             xin_sems, win_sem,
             x0_send, x0_recv, x1_send, x1_recv,
             w_send, w_recv, xs_send, xs_recv, copy_sems):
        my = lax.axis_index("i")
        left = (my - 1) % N_DEV
        right = (my + 1) % N_DEV

        def rcopy(src, dst, ssem, rsem, dev):
            return pltpu.make_async_remote_copy(
                src_ref=src, dst_ref=dst, send_sem=ssem, recv_sem=rsem,
                device_id=(dev,), device_id_type=pl.DeviceIdType.MESH,
            )

        def acc_pair(xpc, wpc, first=False):
            for rb in range(RB):
                rows = pl.ds(rb * rblk, rblk)
                prod = _mm(xpc[rows, :], wpc[...])
                if first:
                    accf[rows, :] = prod.astype(jnp.bfloat16)
                else:
                    accf[rows, :] = (
                        accf[rows, :].astype(jnp.float32) + prod
                    ).astype(jnp.bfloat16)

        barrier = pltpu.get_barrier_semaphore()
        for nbr in (left, right):
            pl.semaphore_signal(
                barrier, inc=1,
                device_id=(nbr,), device_id_type=pl.DeviceIdType.MESH,
            )
        pl.semaphore_wait(barrier, 2)

        with jax.named_scope("cast_inject"):
            xdmas = [None] * XC
            for c in range(min(2, XC)):
                xdmas[c] = pltpu.make_async_copy(
                    x_hbm.at[pl.ds(c * xcb, xcb), :], xstg.at[c % 2],
                    xin_sems.at[c % 2],
                )
                xdmas[c].start()
            wdma = pltpu.make_async_copy(w_hbm, wstg, win_sem)
            wdma.start()
            h0x = []
            for c in range(XC):
                xdmas[c].wait()
                xl[pl.ds(c * xcb, xcb), :] = (
                    xstg[c % 2].astype(jnp.float8_e5m2)
                )
                if c + 2 < XC:
                    xdmas[c + 2] = pltpu.make_async_copy(
                        x_hbm.at[pl.ds((c + 2) * xcb, xcb), :],
                        xstg.at[c % 2], xin_sems.at[c % 2],
                    )
                    xdmas[c + 2].start()
                if c % 2 == 1:
                    s = c // 2
                    srows = pl.ds(s * sblk, sblk)
                    r = rcopy(xl.at[srows, :], xg.at[0, srows, :],
                              x0_send.at[s], x0_recv.at[s], right)
                    r.start()
                    h0x.append(r)
            wdma.wait()
            wl[...] = wstg[...].astype(jnp.float8_e5m2)
            h0w = rcopy(wl, wg.at[0], w_send.at[0], w_recv.at[0], left)
            h0w.start()

        with jax.named_scope("local_pair"):
            acc_pair(xl, wl, first=True)

        with jax.named_scope("ring_fwd"):
            h0w.wait()
            h1w = rcopy(wg.at[0], wg.at[1], w_send.at[1], w_recv.at[1], left)
            h1w.start()
            for r in h0x:
                r.wait()
            h1x = rcopy(xg.at[0], xg.at[1], x1_send, x1_recv, right)
            h1x.start()
            h1w.wait()
            h2w = rcopy(wg.at[1], wg.at[2], w_send.at[2], w_recv.at[2], left)
            h2w.start()
            h2w.wait()

        with jax.named_scope("pair_m1"):
            acc_pair(xg.at[0], wg.at[2])

        with jax.named_scope("wait_h1x"):
            h1x.wait()
        subs = []
        for s in range(SUB):
            srows = pl.ds(s * sblk, sblk)
            r = rcopy(xg.at[1, srows, :], xg.at[2, srows, :],
                      xs_send.at[s], xs_recv.at[s], right)
            r.start()
            subs.append(r)

        with jax.named_scope("pair_m2"):
            acc_pair(xg.at[1], wg.at[1])

        scale = sx_ref[0] * sw_ref[0]
        copies = [None, None]
        with jax.named_scope("stream_m3"):
            for s in range(SUB):
                subs[s].wait()
                for half in range(sblk // rblk):
                    rbi = s * (sblk // rblk) + half
                    rows = pl.ds(rbi * rblk, rblk)
                    val = (
                        accf[rows, :].astype(jnp.float32)
                        + _mm(xg[2, rows, :], wg[0])
                    ) * scale
                    slot = rbi % 2
                    if copies[slot] is not None:
                        copies[slot].wait()
                    stage[slot, :, :] = val
                    cp = pltpu.make_async_copy(
                        stage.at[slot], out_ref.at[rows, :], copy_sems.at[slot]
                    )
                    cp.start()
                    copies[slot] = cp
            for cp in copies:
                cp.wait()

    return pl.pallas_call(
        body,
        out_shape=jax.ShapeDtypeStruct((m, n), jnp.float32),
        in_specs=[
            pl.BlockSpec(memory_space=pl.ANY),
            pl.BlockSpec(memory_space=pl.ANY),
            pl.BlockSpec(memory_space=pltpu.SMEM),
            pl.BlockSpec(memory_space=pltpu.SMEM),
        ],
        out_specs=pl.BlockSpec(memory_space=pltpu.MemorySpace.HBM),
        scratch_shapes=[
            pltpu.VMEM((m, k_per), jnp.float8_e5m2),
            pltpu.VMEM((k_per, n), jnp.float8_e5m2),
            pltpu.VMEM((N_DEV - 1, m, k_per), jnp.float8_e5m2),
            pltpu.VMEM((N_DEV - 1, k_per, n), jnp.float8_e5m2),
            pltpu.VMEM((m, n), jnp.bfloat16),
            pltpu.VMEM((2, rblk, n), jnp.float32),
            pltpu.VMEM((2, xcb, k_per), jnp.float32),
            pltpu.VMEM((k_per, n), jnp.float32),
            pltpu.SemaphoreType.DMA((2,)),
            pltpu.SemaphoreType.DMA(()),
            pltpu.SemaphoreType.DMA((SUB,)),
            pltpu.SemaphoreType.DMA((SUB,)),
            pltpu.SemaphoreType.DMA(()),
            pltpu.SemaphoreType.DMA(()),
            pltpu.SemaphoreType.DMA((N_DEV - 1,)),
            pltpu.SemaphoreType.DMA((N_DEV - 1,)),
            pltpu.SemaphoreType.DMA((SUB,)),
            pltpu.SemaphoreType.DMA((SUB,)),
            pltpu.SemaphoreType.DMA((2,)),
        ],
        compiler_params=pltpu.CompilerParams(
            collective_id=0,
            vmem_limit_bytes=63 * 1024 * 1024,
        ),
    )(x, w_mat, scale_x, scale_w)


# device time: 152139 ns/iter; 1.4860x vs baseline; 1.1507x over previous
import jax
import jax.numpy as jnp
from jax import lax
from jax.experimental import pallas as pl
from jax.experimental.pallas import tpu as pltpu

N_DEV = 4


def kernel(x, w_mat, scale_x, scale_w):
    m, k_per = x.shape
    _, n = w_mat.shape
    mh = m // 2
    nh = n // 2
    cb = 512
    sb = 1024
    f8 = jnp.float8_e5m2

    def _mm(a, b):
        return jax.lax.dot_general(
            a, b,
            dimension_numbers=(((1,), (0,)), ((), ())),
            preferred_element_type=jnp.float32,
        )

    def body(x_hbm, w_hbm, sx_ref, sw_ref, out_ref,
             xla, xlb, wla, wlb, xga, xgb, wga, wgb, accf, stage, xstg, wstg,
             xin_sems, win_sem,
             ra0_s, ra0_r, rw_s, rw_r, ra1_s, ra1_r, ra2_s, ra2_r,
             lb0_s, lb0_r, lw_s, lw_r, lb1_s, lb1_r, lb2_s, lb2_r,
             copy_sems):
        my = lax.axis_index("i")
        left = (my - 1) % N_DEV
        right = (my + 1) % N_DEV

        def rcopy(src, dst, ssem, rsem, dev):
            return pltpu.make_async_remote_copy(
                src_ref=src, dst_ref=dst, send_sem=ssem, recv_sem=rsem,
                device_id=(dev,), device_id_type=pl.DeviceIdType.MESH,
            )

        def quad(dst_rows, dst_cols, xpc, wpc, first=False, blk=512):
            rows_total = xpc.shape[0]
            for r0 in range(0, rows_total, blk):
                rows_in = pl.ds(r0, blk)
                orow = pl.ds(dst_rows + r0, blk)
                ocol = pl.ds(dst_cols, wpc.shape[1])
                prod = _mm(xpc[rows_in, :], wpc[...])
                if first:
                    accf[orow, ocol] = prod.astype(jnp.bfloat16)
                else:
                    accf[orow, ocol] = (
                        accf[orow, ocol].astype(jnp.float32) + prod
                    ).astype(jnp.bfloat16)

        barrier = pltpu.get_barrier_semaphore()
        for nbr in (left, right):
            pl.semaphore_signal(
                barrier, inc=1,
                device_id=(nbr,), device_id_type=pl.DeviceIdType.MESH,
            )
        pl.semaphore_wait(barrier, 2)

        order = [0, 4, 1, 5, 2, 6, 3, 7]
        with jax.named_scope("cast_inject"):
            wdma = pltpu.make_async_copy(w_hbm, wstg, win_sem)
            wdma.start()
            xdmas = {}
            for pos in range(2):
                ci = order[pos]
                xdmas[ci] = pltpu.make_async_copy(
                    x_hbm.at[pl.ds(ci * cb, cb), :], xstg.at[pos % 2],
                    xin_sems.at[pos % 2],
                )
                xdmas[ci].start()
            h0 = []
            w_sent = [False]

            def maybe_send_w():
                if w_sent[0]:
                    return
                w_sent[0] = True
                wdma.wait()
                wla[...] = wstg[:, 0:nh].astype(f8)
                wlb[...] = wstg[:, nh:n].astype(f8)
                rw0 = rcopy(wla, wga.at[0], rw_s.at[0], rw_r.at[0], right)
                lw0 = rcopy(wlb, wgb.at[0], lw_s.at[0], lw_r.at[0], left)
                rw0.start()
                lw0.start()
                h0.extend([rw0, lw0])

            for pos in range(8):
                ci = order[pos]
                xdmas[ci].wait()
                dst = xla if ci < 4 else xlb
                lrow = (ci % 4) * cb
                dst[pl.ds(lrow, cb), :] = xstg[pos % 2].astype(f8)
                if pos + 2 < 8:
                    nci = order[pos + 2]
                    xdmas[nci] = pltpu.make_async_copy(
                        x_hbm.at[pl.ds(nci * cb, cb), :], xstg.at[pos % 2],
                        xin_sems.at[pos % 2],
                    )
                    xdmas[nci].start()
                if ci in (1, 3):
                    s = ci // 2
                    srows = pl.ds(s * sb, sb)
                    r = rcopy(xla.at[srows, :], xga.at[0, srows, :],
                              ra0_s.at[s], ra0_r.at[s], right)
                    r.start()
                    h0.append(r)
                elif ci in (5, 7):
                    s = (ci - 4) // 2
                    srows = pl.ds(s * sb, sb)
                    r = rcopy(xlb.at[srows, :], xgb.at[0, srows, :],
                              lb0_s.at[s], lb0_r.at[s], left)
                    r.start()
                    h0.append(r)
                if pos >= 3:
                    maybe_send_w()
            maybe_send_w()

        with jax.named_scope("local_pair"):
            quad(0, 0, xla, wla, first=True)
            quad(0, nh, xla, wlb, first=True)
            quad(mh, 0, xlb, wla, first=True)
            quad(mh, nh, xlb, wlb, first=True)

        with jax.named_scope("hop1_fwd"):
            for r in h0:
                r.wait()
            rw1 = rcopy(wga.at[0], wga.at[1], rw_s.at[1], rw_r.at[1], right)
            ra1 = rcopy(xga.at[0], xga.at[1], ra1_s, ra1_r, right)
            lw1 = rcopy(wgb.at[0], wgb.at[1], lw_s.at[1], lw_r.at[1], left)
            lb1 = rcopy(xgb.at[0], xgb.at[1], lb1_s, lb1_r, left)
            rw1.start()
            ra1.start()
            lw1.start()
            lb1.start()

        with jax.named_scope("pair_near_same_ring"):
            quad(0, 0, xga.at[0], wga.at[0])
            quad(mh, nh, xgb.at[0], wgb.at[0])

        with jax.named_scope("hop2_start"):
            rw1.wait()
            ra1.wait()
            rw2 = rcopy(wga.at[1], wga.at[2], rw_s.at[2], rw_r.at[2], right)
            rw2.start()
            ra2 = []
            for s in range(2):
                srows = pl.ds(s * sb, sb)
                rr = rcopy(xga.at[1, srows, :], xga.at[2, srows, :],
                           ra2_s.at[s], ra2_r.at[s], right)
                rr.start()
                ra2.append(rr)
            lw1.wait()
            lb1.wait()
            lw2 = rcopy(wgb.at[1], wgb.at[2], lw_s.at[2], lw_r.at[2], left)
            lw2.start()
            lb2 = []
            for s in range(2):
                srows = pl.ds(s * sb, sb)
                rr = rcopy(xgb.at[1, srows, :], xgb.at[2, srows, :],
                           lb2_s.at[s], lb2_r.at[s], left)
                rr.start()
                lb2.append(rr)

        with jax.named_scope("pair_m2"):
            quad(0, 0, xga.at[1], wga.at[1])
            quad(0, nh, xga.at[1], wgb.at[1])
            quad(mh, 0, xgb.at[1], wga.at[1])
            quad(mh, nh, xgb.at[1], wgb.at[1])

        scale = sx_ref[0] * sw_ref[0]
        copies = [None, None]

        def writeout(row0, nrows):
            for r0 in range(row0, row0 + nrows, 512):
                rows = pl.ds(r0, 512)
                slot = (r0 // 512) % 2
                if copies[slot] is not None:
                    copies[slot].wait()
                stage[slot, :, :] = accf[rows, :].astype(jnp.float32) * scale
                cp = pltpu.make_async_copy(
                    stage.at[slot], out_ref.at[rows, :], copy_sems.at[slot]
                )
                cp.start()
                copies[slot] = cp

        with jax.named_scope("tail_stream"):
            rw2.wait()
            quad(mh, 0, xgb.at[0], wga.at[2])
            lw2.wait()
            quad(0, nh, xga.at[0], wgb.at[2])
            for s in range(2):
                ra2[s].wait()
                srows = pl.ds(s * sb, sb)
                quad(s * sb, 0, xga.at[2, srows, :], wga.at[2], blk=512)
                quad(s * sb, nh, xga.at[2, srows, :], wgb.at[0], blk=512)
                writeout(s * sb, sb)
            for s in range(2):
                lb2[s].wait()
                srows = pl.ds(s * sb, sb)
                quad(mh + s * sb, 0, xgb.at[2, srows, :], wga.at[0], blk=512)
                quad(mh + s * sb, nh, xgb.at[2, srows, :], wgb.at[2], blk=512)
                writeout(mh + s * sb, sb)
            for cp in copies:
                cp.wait()

    return pl.pallas_call(
        body,
        out_shape=jax.ShapeDtypeStruct((m, n), jnp.float32),
        in_specs=[
            pl.BlockSpec(memory_space=pl.ANY),
            pl.BlockSpec(memory_space=pl.ANY),
            pl.BlockSpec(memory_space=pltpu.SMEM),
            pl.BlockSpec(memory_space=pltpu.SMEM),
        ],
        out_specs=pl.BlockSpec(memory_space=pltpu.MemorySpace.HBM),
        scratch_shapes=[
            pltpu.VMEM((mh, k_per), f8),
            pltpu.VMEM((mh, k_per), f8),
            pltpu.VMEM((k_per, nh), f8),
            pltpu.VMEM((k_per, nh), f8),
            pltpu.VMEM((N_DEV - 1, mh, k_per), f8),
            pltpu.VMEM((N_DEV - 1, mh, k_per), f8),
            pltpu.VMEM((N_DEV - 1, k_per, nh), f8),
            pltpu.VMEM((N_DEV - 1, k_per, nh), f8),
            pltpu.VMEM((m, n), jnp.bfloat16),
            pltpu.VMEM((2, 512, n), jnp.float32),
            pltpu.VMEM((2, cb, k_per), jnp.float32),
            pltpu.VMEM((k_per, n), jnp.float32),
            pltpu.SemaphoreType.DMA((2,)),
            pltpu.SemaphoreType.DMA(()),
            pltpu.SemaphoreType.DMA((2,)),
            pltpu.SemaphoreType.DMA((2,)),
            pltpu.SemaphoreType.DMA((3,)),
            pltpu.SemaphoreType.DMA((3,)),
            pltpu.SemaphoreType.DMA(()),
            pltpu.SemaphoreType.DMA(()),
            pltpu.SemaphoreType.DMA((2,)),
            pltpu.SemaphoreType.DMA((2,)),
            pltpu.SemaphoreType.DMA((2,)),
            pltpu.SemaphoreType.DMA((2,)),
            pltpu.SemaphoreType.DMA((3,)),
            pltpu.SemaphoreType.DMA((3,)),
            pltpu.SemaphoreType.DMA(()),
            pltpu.SemaphoreType.DMA(()),
            pltpu.SemaphoreType.DMA((2,)),
            pltpu.SemaphoreType.DMA((2,)),
            pltpu.SemaphoreType.DMA((2,)),
        ],
        compiler_params=pltpu.CompilerParams(
            collective_id=0,
            vmem_limit_bytes=63 * 1024 * 1024,
        ),
    )(x, w_mat, scale_x, scale_w)
